# baseline (speedup 1.0000x reference)
import jax
import jax.numpy as jnp
from jax.experimental import pallas as pl


def _cheb(x, edge_index, edge_attr, W, b):
    row, col = edge_index[0], edge_index[1]
    n = x.shape[0]
    deg = jax.ops.segment_sum(edge_attr, row, num_segments=n)
    safe_deg = jnp.where(deg > 0, deg, 1.0)
    dinv = jnp.where(deg > 0, 1.0 / jnp.sqrt(safe_deg), 0.0)
    norm = -dinv[row] * edge_attr * dinv[col]

    def prop(h):
        return jax.ops.segment_sum(norm[:, None] * h[col], row, num_segments=n)

    Tx0 = x
    out = Tx0 @ W[0]
    Tx1 = prop(Tx0)
    out = out + Tx1 @ W[1]
    for k in range(2, W.shape[0]):
        Tx2 = 2.0 * prop(Tx1) - Tx0
        out = out + Tx2 @ W[k]
        Tx0, Tx1 = Tx1, Tx2
    return out + b


def kernel(x, edge_index, edge_attr, batch, W1, b1, W2, b2, W3, b3, Wl, bl):
    G = 8
    h = jax.nn.relu(_cheb(x, edge_index, edge_attr, W1, b1))
    h = jax.nn.relu(_cheb(h, edge_index, edge_attr, W2, b2))
    h = jax.nn.relu(_cheb(h, edge_index, edge_attr, W3, b3))
    sums = jax.ops.segment_sum(h, batch, num_segments=G)
    cnt = jax.ops.segment_sum(jnp.ones((h.shape[0],), h.dtype), batch, num_segments=G)
    hp = sums / jnp.clip(cnt, 1.0)[:, None]
    out = hp @ Wl + bl
    return (out, hp)


# speedup vs baseline: 7.8147x; 7.8147x over previous
import functools

import jax
import jax.numpy as jnp
from jax import lax
from jax.experimental import pallas as pl
from jax.experimental.pallas import tpu as pltpu
from jax.experimental.pallas import tpu_sc as plsc

N = 10000
E = 320000
G = 8
NC, NS, L = 2, 16, 16
NW = NC * NS
CHUNK = 128
CPW = 80
EPW = CHUNK * CPW
EP = EPW * NW
RPT = N // NS

_MESH = plsc.VectorSubcoreMesh(core_axis_name="c", subcore_axis_name="s")


def _bc16(s):
    return jnp.broadcast_to(s, (16,))


def _deg_body(attr3, row3, deg_out, a_v, r_v, zb_v, sacc):
    cid = lax.axis_index("c")
    sid = lax.axis_index("s")

    def zb_body(t, _):
        zb_v[pl.ds(t * 16, 16)] = jnp.zeros((16,), jnp.float32)
        return 0

    lax.fori_loop(0, 40, zb_body, 0)
    pltpu.sync_copy(zb_v, sacc.at[pl.ds(sid * 640, 640)])
    plsc.subcore_barrier()

    for w_off in range(2):
        w = sid * 2 + w_off
        pltpu.sync_copy(attr3.at[w], a_v)
        pltpu.sync_copy(row3.at[w], r_v)

        def deg_body(j, _):
            pltpu.sync_copy(a_v.at[j], sacc.at[r_v.at[j]], add=True)
            return 0

        lax.fori_loop(0, CPW, deg_body, 0)
    plsc.subcore_barrier()

    @pl.when(cid == 0)
    def _():
        pltpu.sync_copy(
            sacc.at[pl.ds(sid * 640, 640)], deg_out.at[pl.ds(sid * 640, 640)]
        )


def _deg(attr3, row3):
    return pl.kernel(
        _deg_body,
        out_type=jax.ShapeDtypeStruct((EPW,), jnp.float32),
        mesh=_MESH,
        scratch_types=[
            pltpu.VMEM((CPW, CHUNK), jnp.float32),
            pltpu.VMEM((CPW, CHUNK), jnp.int32),
            pltpu.VMEM((640,), jnp.float32),
            pltpu.VMEM_SHARED((EPW,), jnp.float32),
        ],
    )(attr3, row3)


def _dinv_body(deg_ref, dinv_ref):
    deg = deg_ref[...]
    dinv_ref[...] = jnp.where(
        deg > 0.0, lax.rsqrt(jnp.where(deg > 0.0, deg, 1.0)), 0.0
    )


def _dinv(deg):
    return pl.pallas_call(
        _dinv_body,
        out_shape=jax.ShapeDtypeStruct((EPW,), jnp.float32),
    )(deg)


def _prep_body(dinv_hbm, attr3, row3, col3, norm3, a_v, r_v, c_v, n_v,
               dr_v, dc_v):
    cid = lax.axis_index("c")
    sid = lax.axis_index("s")

    wid = sid * NC + cid
    pltpu.sync_copy(row3.at[wid], r_v)
    pltpu.sync_copy(col3.at[wid], c_v)
    pltpu.sync_copy(attr3.at[wid], a_v)

    def norm_body(j, _):
        pltpu.sync_copy(dinv_hbm.at[r_v.at[j]], dr_v)
        pltpu.sync_copy(dinv_hbm.at[c_v.at[j]], dc_v)
        for k in range(8):
            sl = pl.ds(k * 16, 16)
            n_v[j, sl] = -(dr_v[sl] * a_v[j, sl] * dc_v[sl])
        return 0

    lax.fori_loop(0, CPW, norm_body, 0)
    pltpu.sync_copy(n_v, norm3.at[wid])


def _prep(dinv, attr3, row3, col3):
    return pl.kernel(
        _prep_body,
        out_type=jax.ShapeDtypeStruct((NW, CPW, CHUNK), jnp.float32),
        mesh=_MESH,
        scratch_types=[
            pltpu.VMEM((CPW, CHUNK), jnp.float32),
            pltpu.VMEM((CPW, CHUNK), jnp.int32),
            pltpu.VMEM((CPW, CHUNK), jnp.int32),
            pltpu.VMEM((CPW, CHUNK), jnp.float32),
            pltpu.VMEM((CHUNK,), jnp.float32),
            pltpu.VMEM((CHUNK,), jnp.float32),
        ],
    )(dinv, attr3, row3, col3)


def _make_prop_body(fw):
    f16 = fw // 16

    def body(y, col3, row3, norm3, out, col_v, row_v, norm_v, rows_v, acc):
        cid = lax.axis_index("c")
        sid = lax.axis_index("s")
        wid = sid * NC + cid
        pltpu.sync_copy(col3.at[wid], col_v)
        pltpu.sync_copy(row3.at[wid], row_v)
        pltpu.sync_copy(norm3.at[wid], norm_v)

        def z_body(r, _):
            for k in range(f16):
                rows_v[r, pl.ds(k * 16, 16)] = jnp.zeros((16,), jnp.float32)
            return 0

        lax.fori_loop(0, CHUNK, z_body, 0)
        for m in range(5):
            pltpu.sync_copy(rows_v, acc.at[pl.ds(sid * 624 + m * 128, 128)])
        plsc.subcore_barrier()

        def chunk_body(j, _):
            pltpu.sync_copy(y.at[col_v.at[j]], rows_v)
            for g in range(8):
                n16 = norm_v[j, pl.ds(g * 16, 16)]
                for l in range(16):
                    nv = n16.at[_bc16(l)].get(mode="promise_in_bounds")
                    i = g * 16 + l
                    for k in range(f16):
                        sl = pl.ds(k * 16, 16)
                        rows_v[i, sl] = rows_v[i, sl] * nv
            pltpu.sync_copy(rows_v, acc.at[row_v.at[j]], add=True)
            return 0

        lax.fori_loop(0, CPW, chunk_body, 0)
        plsc.subcore_barrier()

        @pl.when(sid < 15)
        def _():
            pltpu.sync_copy(
                acc.at[pl.ds(sid * 624, 624)],
                out.at[cid, pl.ds(sid * 624, 624)],
            )

        @pl.when(sid == 15)
        def _():
            pltpu.sync_copy(
                acc.at[pl.ds(9360, 640)],
                out.at[cid, pl.ds(9360, 640)],
            )

    return body


@functools.cache
def _prop_call(fw):
    return pl.kernel(
        _make_prop_body(fw),
        out_type=jax.ShapeDtypeStruct((NC, N, fw), jnp.float32),
        mesh=_MESH,
        scratch_types=[
            pltpu.VMEM((CPW, CHUNK), jnp.int32),
            pltpu.VMEM((CPW, CHUNK), jnp.int32),
            pltpu.VMEM((CPW, CHUNK), jnp.float32),
            pltpu.VMEM((CHUNK, fw), jnp.float32),
            pltpu.VMEM_SHARED((N, fw), jnp.float32),
        ],
    )


def _prop(y, col3, row3, norm3):
    return _prop_call(y.shape[1])(y, col3, row3, norm3)


def _tc_a_body(h_ref, pp_ref, wa_ref, wb_ref, t1_ref, m_ref):
    t1 = pp_ref[0] + pp_ref[1]
    t1_ref[...] = t1
    m_ref[...] = jnp.dot(
        h_ref[...], wa_ref[...], preferred_element_type=jnp.float32
    ) + jnp.dot(t1, wb_ref[...], preferred_element_type=jnp.float32)


def _tc_a(h, pp, wa, wb):
    return pl.pallas_call(
        _tc_a_body,
        out_shape=(
            jax.ShapeDtypeStruct((N, 128), jnp.float32),
            jax.ShapeDtypeStruct((N, 128), jnp.float32),
        ),
    )(h, pp, wa, wb)


def _tc_b_body(m_ref, up_ref, wc_ref, b_ref, o_ref):
    u = up_ref[0] + up_ref[1]
    o_ref[...] = jnp.maximum(
        m_ref[...]
        + jnp.dot(u, wc_ref[...], preferred_element_type=jnp.float32)
        + b_ref[...],
        0.0,
    )


def _tc_b(m, up, wc, b):
    return pl.pallas_call(
        _tc_b_body,
        out_shape=jax.ShapeDtypeStruct((N, 128), jnp.float32),
    )(m, up, wc, b)


def _tc_final_body(m_ref, up_ref, wc_ref, b_ref, batch_ref, wl_ref, bl_ref,
                   out_ref, hp_ref):
    u = up_ref[0] + up_ref[1]
    h3 = jnp.maximum(
        m_ref[...]
        + jnp.dot(u, wc_ref[...], preferred_element_type=jnp.float32)
        + b_ref[...],
        0.0,
    )
    bvec = batch_ref[...]
    ids = lax.broadcasted_iota(jnp.int32, (G, N), 0)
    mask = (bvec[None, :] == ids).astype(jnp.float32)
    sums = jnp.dot(mask, h3, preferred_element_type=jnp.float32)
    cnt = jnp.sum(mask, axis=1, keepdims=True)
    hp = sums / jnp.maximum(cnt, 1.0)
    hp_ref[...] = hp
    out_ref[...] = (
        jnp.dot(hp, wl_ref[...], preferred_element_type=jnp.float32)
        + bl_ref[...]
    )


def _tc_final(m, up, wc, b, batch, wl, bl):
    return pl.pallas_call(
        _tc_final_body,
        out_shape=(
            jax.ShapeDtypeStruct((G, 2), jnp.float32),
            jax.ShapeDtypeStruct((G, 128), jnp.float32),
        ),
    )(m, up, wc, b, batch, wl, bl)


def kernel(x, edge_index, edge_attr, batch, W1, b1, W2, b2, W3, b3, Wl, bl):
    row, col = edge_index[0], edge_index[1]
    pad = EP - E
    rng = jnp.arange(pad, dtype=jnp.int32) % N
    row3 = jnp.concatenate([row, rng]).reshape(NW, CPW, CHUNK)
    col3 = jnp.concatenate([col, rng]).reshape(NW, CPW, CHUNK)
    attr3 = jnp.concatenate(
        [edge_attr, jnp.zeros((pad,), jnp.float32)]
    ).reshape(NW, CPW, CHUNK)

    deg = _deg(attr3, row3)
    dinv = _dinv(deg)
    norm3 = _prep(dinv, attr3, row3, col3)

    def pad_w(w):
        return jnp.pad(w, ((0, 128 - w.shape[0]), (0, 128 - w.shape[1])))

    def pad_b(b):
        return jnp.pad(b, (0, 128 - b.shape[0]))

    h = x
    for li, (W, b) in enumerate(((W1, b1), (W2, b2), (W3, b3))):
        wa = pad_w(W[0] - W[2])
        wb = pad_w(W[1])
        wc = pad_w(2.0 * W[2])
        pp = _prop(h, col3, row3, norm3)
        t1, m = _tc_a(h, pp, wa, wb)
        up = _prop(t1, col3, row3, norm3)
        if li == 2:
            out, hp = _tc_final(m, up, wc, pad_b(b), batch, Wl, bl)
            return (out, hp)
        h = _tc_b(m, up, wc, pad_b(b))


# speedup vs baseline: 14.8168x; 1.8960x over previous
import functools

import jax
import jax.numpy as jnp
from jax import lax
from jax.experimental import pallas as pl
from jax.experimental.pallas import tpu as pltpu
from jax.experimental.pallas import tpu_sc as plsc

N = 10000
E = 320000
G = 8
NC, NS, L = 2, 16, 16
NW = NC * NS
CHUNK = 128
CPW = 80
EPW = CHUNK * CPW
CH = 64
CPP = EPW // CH
EP = EPW * NW
RPT = N // NS

_MESH = plsc.VectorSubcoreMesh(core_axis_name="c", subcore_axis_name="s")


def _bc16(s):
    return jnp.broadcast_to(s, (16,))


def _deg_body(attrf, rowf, deg_out, a_v, r_v, zb_v, sacc):
    cid = lax.axis_index("c")
    sid = lax.axis_index("s")

    def zb_body(t, _):
        zb_v[pl.ds(t * 16, 16)] = jnp.zeros((16,), jnp.float32)
        return 0

    lax.fori_loop(0, 40, zb_body, 0)
    pltpu.sync_copy(zb_v, sacc.at[pl.ds(sid * 640, 640)])
    plsc.subcore_barrier()

    for w_off in range(2):
        w = sid * 2 + w_off
        pltpu.sync_copy(attrf.at[pl.ds(w * EPW, EPW)], a_v)
        pltpu.sync_copy(rowf.at[pl.ds(w * EPW, EPW)], r_v)
        pltpu.sync_copy(a_v, sacc.at[r_v], add=True)
    plsc.subcore_barrier()

    @pl.when(cid == 0)
    def _():
        pltpu.sync_copy(
            sacc.at[pl.ds(sid * 640, 640)], deg_out.at[pl.ds(sid * 640, 640)]
        )


def _deg(attrf, rowf):
    return pl.kernel(
        _deg_body,
        out_type=jax.ShapeDtypeStruct((EPW,), jnp.float32),
        mesh=_MESH,
        scratch_types=[
            pltpu.VMEM((EPW,), jnp.float32),
            pltpu.VMEM((EPW,), jnp.int32),
            pltpu.VMEM((640,), jnp.float32),
            pltpu.VMEM_SHARED((EPW,), jnp.float32),
        ],
    )(attrf, rowf)


def _dinv_body(deg_ref, dinv_ref):
    deg = deg_ref[...]
    dinv_ref[...] = jnp.where(
        deg > 0.0, lax.rsqrt(jnp.where(deg > 0.0, deg, 1.0)), 0.0
    )


def _dinv(deg):
    return pl.pallas_call(
        _dinv_body,
        out_shape=jax.ShapeDtypeStruct((EPW,), jnp.float32),
    )(deg)


def _prep_body(dinv_hbm, attrf, rowf, colf, normf, a_v, r_v, c_v, n_v,
               dr_v, dc_v):
    cid = lax.axis_index("c")
    sid = lax.axis_index("s")

    wid = sid * NC + cid
    base = wid * EPW
    pltpu.sync_copy(rowf.at[pl.ds(base, EPW)], r_v)
    pltpu.sync_copy(colf.at[pl.ds(base, EPW)], c_v)
    pltpu.sync_copy(attrf.at[pl.ds(base, EPW)], a_v)

    pltpu.sync_copy(dinv_hbm.at[r_v], dr_v)
    pltpu.sync_copy(dinv_hbm.at[c_v], dc_v)

    def norm_body(j, _):
        sl = pl.ds(j * 16, 16)
        n_v[sl] = -(dr_v[sl] * a_v[sl] * dc_v[sl])
        return 0

    lax.fori_loop(0, EPW // 16, norm_body, 0)
    pltpu.sync_copy(n_v, normf.at[pl.ds(base, EPW)])


def _prep(dinv, attrf, rowf, colf):
    return pl.kernel(
        _prep_body,
        out_type=jax.ShapeDtypeStruct((EP,), jnp.float32),
        mesh=_MESH,
        scratch_types=[
            pltpu.VMEM((EPW,), jnp.float32),
            pltpu.VMEM((EPW,), jnp.int32),
            pltpu.VMEM((EPW,), jnp.int32),
            pltpu.VMEM((EPW,), jnp.float32),
            pltpu.VMEM((EPW,), jnp.float32),
            pltpu.VMEM((EPW,), jnp.float32),
        ],
    )(dinv, attrf, rowf, colf)


def _make_prop_body(fw):
    f16 = fw // 16

    def body(y, colf, rowf, normf, out, col_v,
             d0, d1, d2, d3, nr0, nr1, nr2, nr3, rw0, rw1, rw2, rw3,
             acc, gsem, nsem, ssem):
        d = (d0, d1, d2, d3)
        nr = (nr0, nr1, nr2, nr3)
        rw = (rw0, rw1, rw2, rw3)
        cid = lax.axis_index("c")
        sid = lax.axis_index("s")
        wid = sid * NC + cid
        base = wid * EPW
        pltpu.sync_copy(colf.at[pl.ds(base, EPW)], col_v)

        def g_start(c, b):
            pltpu.async_copy(y.at[col_v.at[pl.ds(c * CH, CH)]], d[b],
                             gsem.at[b])

        def g_wait(b):
            pltpu.make_async_copy(y.at[col_v.at[pl.ds(0, CH)]], d[b],
                                  gsem.at[b]).wait()

        def nr_start(c, b):
            pltpu.async_copy(normf.at[pl.ds(base + c * CH, CH)], nr[b],
                             nsem.at[b])
            pltpu.async_copy(rowf.at[pl.ds(base + c * CH, CH)], rw[b],
                             nsem.at[b])

        def nr_wait(b):
            pltpu.make_async_copy(normf.at[pl.ds(0, CH)], nr[b],
                                  nsem.at[b]).wait()
            pltpu.make_async_copy(rowf.at[pl.ds(0, CH)], rw[b],
                                  nsem.at[b]).wait()

        def s_start(b):
            pltpu.async_copy(d[b], acc.at[rw[b]], ssem.at[b], add=True)

        def s_wait(b):
            pltpu.make_async_copy(d[b], acc.at[rw[b]], ssem.at[b]).wait()

        def scale(b):
            def grp(g, _):
                n16 = nr[b][pl.ds(g * 16, 16)]
                for l in range(16):
                    nv = n16.at[_bc16(l)].get(mode="promise_in_bounds")
                    i = g * 16 + l
                    for k in range(f16):
                        sl = pl.ds(k * 16, 16)
                        d[b][i, sl] = d[b][i, sl] * nv
                return 0

            lax.fori_loop(0, CH // 16, grp, 0)

        for b in range(3):
            nr_start(b, b)
            g_start(b, b)

        def z_body(r, _):
            for k in range(f16):
                d3[r, pl.ds(k * 16, 16)] = jnp.zeros((16,), jnp.float32)
            return 0

        lax.fori_loop(0, CH, z_body, 0)
        for m in range(10):
            pltpu.sync_copy(d3, acc.at[pl.ds(sid * 624 + m * CH, CH)])
        plsc.subcore_barrier()

        def chunk_body(jj, _):
            for b in range(4):
                c = 4 * jj + b
                g_wait(b)
                nr_wait(b)
                scale(b)
                s_start(b)
                nb = (b + 3) % 4

                @pl.when(c + 3 < CPP)
                def _():
                    @pl.when(c >= 1)
                    def _():
                        s_wait(nb)

                    nr_start(c + 3, nb)
                    g_start(c + 3, nb)

            return 0

        lax.fori_loop(0, CPP // 4, chunk_body, 0)
        for b in range(4):
            s_wait(b)
        plsc.subcore_barrier()

        @pl.when(sid < 15)
        def _():
            pltpu.sync_copy(
                acc.at[pl.ds(sid * 624, 624)],
                out.at[cid, pl.ds(sid * 624, 624)],
            )

        @pl.when(sid == 15)
        def _():
            pltpu.sync_copy(
                acc.at[pl.ds(9360, 640)],
                out.at[cid, pl.ds(9360, 640)],
            )

    return body


@functools.cache
def _prop_call(fw):
    return pl.kernel(
        _make_prop_body(fw),
        out_type=jax.ShapeDtypeStruct((NC, N, fw), jnp.float32),
        mesh=_MESH,
        scratch_types=(
            [pltpu.VMEM((EPW,), jnp.int32)]
            + [pltpu.VMEM((CH, fw), jnp.float32)] * 4
            + [pltpu.VMEM((CH,), jnp.float32)] * 4
            + [pltpu.VMEM((CH,), jnp.int32)] * 4
            + [
                pltpu.VMEM_SHARED((N, fw), jnp.float32),
                pltpu.SemaphoreType.DMA((4,)),
                pltpu.SemaphoreType.DMA((4,)),
                pltpu.SemaphoreType.DMA((4,)),
            ]
        ),
    )


def _prop(y, colf, rowf, normf):
    return _prop_call(y.shape[1])(y, colf, rowf, normf)


def _tc_a_body(h_ref, pp_ref, wa_ref, wb_ref, t1_ref, m_ref):
    t1 = pp_ref[0] + pp_ref[1]
    t1_ref[...] = t1
    m_ref[...] = jnp.dot(
        h_ref[...], wa_ref[...], preferred_element_type=jnp.float32
    ) + jnp.dot(t1, wb_ref[...], preferred_element_type=jnp.float32)


def _tc_a(h, pp, wa, wb):
    return pl.pallas_call(
        _tc_a_body,
        out_shape=(
            jax.ShapeDtypeStruct((N, 128), jnp.float32),
            jax.ShapeDtypeStruct((N, 128), jnp.float32),
        ),
    )(h, pp, wa, wb)


def _tc_b_body(m_ref, up_ref, wc_ref, b_ref, o_ref):
    u = up_ref[0] + up_ref[1]
    o_ref[...] = jnp.maximum(
        m_ref[...]
        + jnp.dot(u, wc_ref[...], preferred_element_type=jnp.float32)
        + b_ref[...],
        0.0,
    )


def _tc_b(m, up, wc, b):
    return pl.pallas_call(
        _tc_b_body,
        out_shape=jax.ShapeDtypeStruct((N, 128), jnp.float32),
    )(m, up, wc, b)


def _tc_final_body(m_ref, up_ref, wc_ref, b_ref, batch_ref, wl_ref, bl_ref,
                   out_ref, hp_ref):
    u = up_ref[0] + up_ref[1]
    h3 = jnp.maximum(
        m_ref[...]
        + jnp.dot(u, wc_ref[...], preferred_element_type=jnp.float32)
        + b_ref[...],
        0.0,
    )
    bvec = batch_ref[...]
    ids = lax.broadcasted_iota(jnp.int32, (G, N), 0)
    mask = (bvec[None, :] == ids).astype(jnp.float32)
    sums = jnp.dot(mask, h3, preferred_element_type=jnp.float32)
    cnt = jnp.sum(mask, axis=1, keepdims=True)
    hp = sums / jnp.maximum(cnt, 1.0)
    hp_ref[...] = hp
    out_ref[...] = (
        jnp.dot(hp, wl_ref[...], preferred_element_type=jnp.float32)
        + bl_ref[...]
    )


def _tc_final(m, up, wc, b, batch, wl, bl):
    return pl.pallas_call(
        _tc_final_body,
        out_shape=(
            jax.ShapeDtypeStruct((G, 2), jnp.float32),
            jax.ShapeDtypeStruct((G, 128), jnp.float32),
        ),
    )(m, up, wc, b, batch, wl, bl)


def kernel(x, edge_index, edge_attr, batch, W1, b1, W2, b2, W3, b3, Wl, bl):
    row, col = edge_index[0], edge_index[1]
    pad = EP - E
    rng = jnp.arange(pad, dtype=jnp.int32) % N
    rowf = jnp.concatenate([row, rng])
    colf = jnp.concatenate([col, rng])
    attrf = jnp.concatenate([edge_attr, jnp.zeros((pad,), jnp.float32)])

    deg = _deg(attrf, rowf)
    dinv = _dinv(deg)
    normf = _prep(dinv, attrf, rowf, colf)

    def pad_w(w):
        return jnp.pad(w, ((0, 128 - w.shape[0]), (0, 128 - w.shape[1])))

    def pad_b(b):
        return jnp.pad(b, (0, 128 - b.shape[0]))

    h = x
    for li, (W, b) in enumerate(((W1, b1), (W2, b2), (W3, b3))):
        wa = pad_w(W[0] - W[2])
        wb = pad_w(W[1])
        wc = pad_w(2.0 * W[2])
        pp = _prop(h, colf, rowf, normf)
        t1, m = _tc_a(h, pp, wa, wb)
        up = _prop(t1, colf, rowf, normf)
        if li == 2:
            out, hp = _tc_final(m, up, wc, pad_b(b), batch, Wl, bl)
            return (out, hp)
        h = _tc_b(m, up, wc, pad_b(b))
